# baseline (speedup 1.0000x reference)
import functools

import jax
import jax.numpy as jnp
from jax import lax
from jax.experimental import pallas as pl
from jax.experimental.pallas import tpu as pltpu
from jax.experimental.pallas import tpu_sc as plsc

_NUM_CORES = 2
_NUM_SUBCORES = 16
_LANES = 16
_NW = _NUM_CORES * _NUM_SUBCORES
_CHUNK = 128


@functools.lru_cache(maxsize=None)
def _build_sc_kernel(B: int, D: int):
    assert B % (_NW * _LANES) == 0
    bpw = B // _NW
    groups = bpw // _LANES
    nchunks = bpw // _CHUNK

    mesh = plsc.VectorSubcoreMesh(core_axis_name="c", subcore_axis_name="s")

    @functools.partial(
        pl.kernel,
        mesh=mesh,
        out_type=jax.ShapeDtypeStruct((B,), jnp.float32),
        compiler_params=pltpu.CompilerParams(needs_layout_passes=False),
        scratch_types=[
            pltpu.VMEM((bpw,), jnp.int32),
            pltpu.VMEM((bpw,), jnp.int32),
            pltpu.VMEM((_CHUNK, D), jnp.float32),
            pltpu.VMEM((_CHUNK, D), jnp.float32),
            pltpu.VMEM((_CHUNK, 1), jnp.float32),
            pltpu.VMEM((_CHUNK, 1), jnp.float32),
            pltpu.VMEM((bpw,), jnp.float32),
            pltpu.SemaphoreType.DMA,
            pltpu.SemaphoreType.DMA,
            pltpu.SemaphoreType.DMA,
            pltpu.SemaphoreType.DMA,
        ],
    )
    def body(uid_hbm, iid_hbm, uemb_hbm, iemb_hbm, ubias_hbm, ibias_hbm,
             out_hbm, uid_v, iid_v, urows, irows, ub_v, ib_v,
             out_v, usem, isem, ubsem, ibsem):
        wid = lax.axis_index("s") * _NUM_CORES + lax.axis_index("c")
        base = wid * bpw

        pltpu.sync_copy(uid_hbm.at[pl.ds(base, bpw)], uid_v)
        pltpu.sync_copy(iid_hbm.at[pl.ds(base, bpw)], iid_v)

        def chunk(c, carry):
            cbase = c * _CHUNK

            def fire(g, carry2):
                gb = g * _LANES
                uvec = uid_v[pl.ds(cbase + gb, _LANES)]
                ivec = iid_v[pl.ds(cbase + gb, _LANES)]
                for j in range(_LANES):
                    u = uvec[j]
                    v = ivec[j]
                    i = gb + j
                    pltpu.async_copy(uemb_hbm.at[pl.ds(u, 1), :],
                                     urows.at[pl.ds(i, 1), :], usem)
                    pltpu.async_copy(iemb_hbm.at[pl.ds(v, 1), :],
                                     irows.at[pl.ds(i, 1), :], isem)
                    pltpu.async_copy(ubias_hbm.at[pl.ds(u, 1), :],
                                     ub_v.at[pl.ds(i, 1), :], ubsem)
                    pltpu.async_copy(ibias_hbm.at[pl.ds(v, 1), :],
                                     ib_v.at[pl.ds(i, 1), :], ibsem)
                return carry2

            lax.fori_loop(0, _CHUNK // _LANES, fire, 0)
            pltpu.make_async_copy(
                uemb_hbm.at[pl.ds(0, _CHUNK), :],
                urows, usem).wait()
            pltpu.make_async_copy(
                iemb_hbm.at[pl.ds(0, _CHUNK), :],
                irows, isem).wait()
            pltpu.make_async_copy(
                ubias_hbm.at[pl.ds(0, _CHUNK), :],
                ub_v, ubsem).wait()
            pltpu.make_async_copy(
                ibias_hbm.at[pl.ds(0, _CHUNK), :],
                ib_v, ibsem).wait()

            def group(g, carry2):
                gbase = g * _LANES
                lanes = gbase + lax.iota(jnp.int32, _LANES)
                zeros = jnp.zeros((_LANES,), jnp.int32)
                acc = (plsc.load_gather(ub_v, [lanes, zeros])
                       + plsc.load_gather(ib_v, [lanes, zeros]))
                for d in range(D):
                    col = jnp.full((_LANES,), d, jnp.int32)
                    acc = acc + (plsc.load_gather(urows, [lanes, col])
                                 * plsc.load_gather(irows, [lanes, col]))
                out_v[pl.ds(cbase + gbase, _LANES)] = acc
                return carry2

            lax.fori_loop(0, _CHUNK // _LANES, group, 0)
            return carry

        lax.fori_loop(0, nchunks, chunk, 0)
        pltpu.sync_copy(out_v, out_hbm.at[pl.ds(base, bpw)])

    return body


def kernel(user_id, item_id, user_emb, item_emb, user_bias, item_bias):
    B = user_id.shape[0]
    D = user_emb.shape[1]
    fn = _build_sc_kernel(B, D)
    return fn(
        user_id.astype(jnp.int32),
        item_id.astype(jnp.int32),
        user_emb,
        item_emb,
        user_bias,
        item_bias,
    )


# speedup vs baseline: 1.1485x; 1.1485x over previous
import functools

import jax
import jax.numpy as jnp
from jax import lax
from jax.experimental import pallas as pl
from jax.experimental.pallas import tpu as pltpu
from jax.experimental.pallas import tpu_sc as plsc

_NUM_CORES = 2
_NUM_SUBCORES = 16
_LANES = 16
_NW = _NUM_CORES * _NUM_SUBCORES
_CHUNK = 128
_PACK = 4


@functools.lru_cache(maxsize=None)
def _build_sc_kernel(B: int, D: int):
    assert B % (_NW * _CHUNK) == 0
    bpw = B // _NW
    nchunks = bpw // _CHUNK
    width = D * _PACK

    mesh = plsc.VectorSubcoreMesh(core_axis_name="c", subcore_axis_name="s")

    @functools.partial(
        pl.kernel,
        mesh=mesh,
        out_type=jax.ShapeDtypeStruct((B,), jnp.float32),
        compiler_params=pltpu.CompilerParams(needs_layout_passes=False),
        scratch_types=[
            pltpu.VMEM((bpw,), jnp.int32),
            pltpu.VMEM((bpw,), jnp.int32),
            pltpu.VMEM((_CHUNK,), jnp.int32),
            pltpu.VMEM((_CHUNK,), jnp.int32),
            pltpu.VMEM((_CHUNK, D * _PACK), jnp.float32),
            pltpu.VMEM((_CHUNK, D * _PACK), jnp.float32),
            pltpu.VMEM((_CHUNK,), jnp.float32),
            pltpu.VMEM((_CHUNK,), jnp.float32),
            pltpu.VMEM((bpw,), jnp.float32),
            pltpu.SemaphoreType.DMA,
            pltpu.SemaphoreType.DMA,
            pltpu.SemaphoreType.DMA,
            pltpu.SemaphoreType.DMA,
        ],
    )
    def body(uid_hbm, iid_hbm, uemb_hbm, iemb_hbm, ubias_hbm, ibias_hbm,
             out_hbm, uid_v, iid_v, ublk_v, iblk_v, urows, irows, ub_v, ib_v,
             out_v, usem, isem, ubsem, ibsem):
        wid = lax.axis_index("s") * _NUM_CORES + lax.axis_index("c")
        base = wid * bpw

        pltpu.sync_copy(uid_hbm.at[pl.ds(base, bpw)], uid_v)
        pltpu.sync_copy(iid_hbm.at[pl.ds(base, bpw)], iid_v)

        def chunk(c, carry):
            cbase = c * _CHUNK

            def blkids(g, carry2):
                gb = g * _LANES
                uvec = uid_v[pl.ds(cbase + gb, _LANES)]
                ivec = iid_v[pl.ds(cbase + gb, _LANES)]
                ublk_v[pl.ds(gb, _LANES)] = lax.shift_right_logical(uvec, 2)
                iblk_v[pl.ds(gb, _LANES)] = lax.shift_right_logical(ivec, 2)
                return carry2

            lax.fori_loop(0, _CHUNK // _LANES, blkids, 0)

            pltpu.async_copy(uemb_hbm.at[ublk_v], urows, usem)
            pltpu.async_copy(iemb_hbm.at[iblk_v], irows, isem)
            uix = uid_v.at[pl.ds(cbase, _CHUNK)]
            iix = iid_v.at[pl.ds(cbase, _CHUNK)]
            pltpu.async_copy(ubias_hbm.at[uix], ub_v, ubsem)
            pltpu.async_copy(ibias_hbm.at[iix], ib_v, ibsem)

            pltpu.make_async_copy(
                uemb_hbm.at[pl.ds(0, _CHUNK), :], urows, usem).wait()
            pltpu.make_async_copy(
                iemb_hbm.at[pl.ds(0, _CHUNK), :], irows, isem).wait()
            pltpu.make_async_copy(
                ubias_hbm.at[pl.ds(0, _CHUNK)], ub_v, ubsem).wait()
            pltpu.make_async_copy(
                ibias_hbm.at[pl.ds(0, _CHUNK)], ib_v, ibsem).wait()

            def group(g, carry2):
                gbase = g * _LANES
                lanes = gbase + lax.iota(jnp.int32, _LANES)
                uvec = uid_v[pl.ds(cbase + gbase, _LANES)]
                ivec = iid_v[pl.ds(cbase + gbase, _LANES)]
                uoff = lax.shift_left(
                    lax.bitwise_and(uvec, jnp.int32(_PACK - 1)),
                    jnp.int32(5))
                ioff = lax.shift_left(
                    lax.bitwise_and(ivec, jnp.int32(_PACK - 1)),
                    jnp.int32(5))
                acc = ub_v[pl.ds(gbase, _LANES)] + ib_v[pl.ds(gbase, _LANES)]
                for d in range(D):
                    acc = acc + (plsc.load_gather(urows, [lanes, uoff + d])
                                 * plsc.load_gather(irows, [lanes, ioff + d]))
                out_v[pl.ds(cbase + gbase, _LANES)] = acc
                return carry2

            lax.fori_loop(0, _CHUNK // _LANES, group, 0)
            return carry

        lax.fori_loop(0, nchunks, chunk, 0)
        pltpu.sync_copy(out_v, out_hbm.at[pl.ds(base, bpw)])

    return body


def kernel(user_id, item_id, user_emb, item_emb, user_bias, item_bias):
    B = user_id.shape[0]
    N, D = user_emb.shape
    fn = _build_sc_kernel(B, D)
    return fn(
        user_id.astype(jnp.int32),
        item_id.astype(jnp.int32),
        user_emb.reshape(N // _PACK, D * _PACK),
        item_emb.reshape(N // _PACK, D * _PACK),
        user_bias.reshape(-1),
        item_bias.reshape(-1),
    )
